# Initial kernel scaffold; baseline (speedup 1.0000x reference)
import jax
import jax.numpy as jnp
from jax.experimental import pallas as pl


def kernel(uv, curvatures, edge_feat, tri_feat, edge_index, tri_edges, rows0, cols0, vals0, rows1, cols1, vals1, rows2, cols2, vals2, enc_w1, enc_b1, enc_w2, enc_b2, nu_w1, nu_b1, nu_w2, nu_b2, eu_w1, eu_b1, eu_w2, eu_b2, tu_w1, tu_b1, tu_w2, tu_b2, e2n_w, e2n_b, n2e_w, n2e_b, t2e_w, t2e_b, e2t_w, e2t_b, dec_w1, dec_b1, dec_w2, dec_b2):
    raise NotImplementedError("write your pallas kernel here")


# speedup vs baseline: 1.0516x; 1.0516x over previous
import functools

import jax
import jax.numpy as jnp
from jax.experimental import pallas as pl
from jax.experimental.pallas import tpu as pltpu

N = 50000
E = 400000
T = 200000
H = 32
L = 3


def _dense_call(fn, rows, bn, row_ins, full_ins, out_w):
    grid = rows // bn
    in_specs = (
        [pl.BlockSpec((bn, x.shape[-1]), lambda i: (i, 0)) for x in row_ins]
        + [pl.BlockSpec(x.shape, lambda i, nd=x.ndim: (0,) * nd) for x in full_ins]
    )
    return pl.pallas_call(
        fn,
        grid=(grid,),
        in_specs=in_specs,
        out_specs=pl.BlockSpec((bn, out_w), lambda i: (i, 0)),
        out_shape=jax.ShapeDtypeStruct((rows, out_w), jnp.float32),
    )(*row_ins, *full_ins)


def _enc_body(uv, curv, w1, b1, w2, b2, out):
    h = jnp.tanh(uv[...] @ w1[0:2, :] + curv[...] @ w1[2:4, :] + b1[...])
    out[...] = jnp.tanh(h @ w2[...] + b2[...])


def _node_body(nf, nsm, accD, ndeg4, e2nw, e2nb, w1a, w1b, b1, w2, b2, out):
    cnt = ndeg4[...][:, 0:1]
    nfe = (accD[...] @ e2nw[...] + cnt * e2nb[...]) / jnp.maximum(cnt, 1.0)
    t = jnp.tanh(nf[...] @ w1a[...] + nfe @ w1b[...] + b1[...])
    out[...] = nf[...] + 0.1 * (t @ w2[...] + b2[...]) - 0.05 * nsm[...]


def _edge_body(ef, esm, rawE, accC, tdeg4, n2ew, n2eb, t2ew, t2eb,
               w1a, w1b, b1, w2, b2, out):
    cnt = tdeg4[...][:, 0:1]
    efn = rawE[...] @ n2ew[...] + n2eb[...]
    eft = (accC[...] @ t2ew[...] + cnt * t2eb[...]) / jnp.maximum(cnt, 1.0)
    m = efn + eft
    t = jnp.tanh(ef[...] @ w1a[...] + m @ w1b[...] + b1[...])
    out[...] = ef[...] + 0.1 * (t @ w2[...] + b2[...]) - 0.05 * esm[...]


def _tri_body(tf, tsm, rawF, e2tw, e2tb, w1a, w1b, b1, w2, b2, out):
    tfe = rawF[...] @ e2tw[...] + e2tb[...]
    t = jnp.tanh(tf[...] @ w1a[...] + tfe @ w1b[...] + b1[...])
    out[...] = tf[...] + 0.1 * (t @ w2[...] + b2[...]) - 0.05 * tsm[...]


def _dec_body(nf, uv, w1, b1, w2, b2, out):
    lap = jnp.tanh(nf[...] @ w1[...] + b1[...]) @ w2[...] + b2[...]
    u = uv[...][:, 0:1]
    v = uv[...][:, 1:2]
    ru = (0.1 - u) * (u - 1.0) * u - v
    rv = 0.01 * (0.5 * u - v)
    out[...] = jnp.concatenate(
        [ru + 1e-4 * lap[:, 0:1], rv + 1e-5 * lap[:, 1:2]], axis=-1)


def _spmm_jax(rows, cols, vals, x, n):
    return jnp.zeros((n, x.shape[1]), x.dtype).at[rows].add(vals[:, None] * x[cols])


def kernel(uv, curvatures, edge_feat, tri_feat, edge_index, tri_edges,
           rows0, cols0, vals0, rows1, cols1, vals1, rows2, cols2, vals2,
           enc_w1, enc_b1, enc_w2, enc_b2, nu_w1, nu_b1, nu_w2, nu_b2,
           eu_w1, eu_b1, eu_w2, eu_b2, tu_w1, tu_b1, tu_w2, tu_b2,
           e2n_w, e2n_b, n2e_w, n2e_b, t2e_w, t2e_b, e2t_w, e2t_b,
           dec_w1, dec_b1, dec_w2, dec_b2):
    src = edge_index[0]
    dst = edge_index[1]
    te0 = tri_edges[:, 0]
    te1 = tri_edges[:, 1]
    te2 = tri_edges[:, 2]

    ndeg4 = (jnp.zeros((N, 4), jnp.float32).at[src].add(1.0).at[dst].add(1.0))
    tdeg4 = (jnp.zeros((E, 4), jnp.float32)
             .at[te0].add(1.0).at[te1].add(1.0).at[te2].add(1.0))

    nf = _dense_call(_enc_body, N, 2000, [uv, curvatures],
                     [enc_w1, enc_b1.reshape(1, H), enc_w2, enc_b2.reshape(1, H)], H)
    ef = edge_feat
    tf = tri_feat

    for l in range(L):
        nsm = _spmm_jax(rows0, cols0, vals0, nf, N)
        esm = _spmm_jax(rows1, cols1, vals1, ef, E)
        tsm = _spmm_jax(rows2, cols2, vals2, tf, T)
        accD = jnp.zeros((N, 4), jnp.float32).at[src].add(ef).at[dst].add(ef)
        rawE = 0.5 * (nf[src] + nf[dst])
        accC = (jnp.zeros((E, 4), jnp.float32)
                .at[te0].add(tf).at[te1].add(tf).at[te2].add(tf))
        rawF = (ef[te0] + ef[te1] + ef[te2]) * (1.0 / 3.0)

        nf = _dense_call(
            _node_body, N, 2000, [nf, nsm, accD, ndeg4],
            [e2n_w[l], e2n_b[l].reshape(1, H),
             nu_w1[l][:H], nu_w1[l][H:], nu_b1[l].reshape(1, H),
             nu_w2[l], nu_b2[l].reshape(1, H)], H)
        ef = _dense_call(
            _edge_body, E, 2000, [ef, esm, rawE, accC, tdeg4],
            [n2e_w[l], n2e_b[l].reshape(1, H), t2e_w[l], t2e_b[l].reshape(1, H),
             eu_w1[l][:4], eu_w1[l][4:], eu_b1[l].reshape(1, H),
             eu_w2[l], eu_b2[l].reshape(1, 4)], 4)
        tf = _dense_call(
            _tri_body, T, 2000, [tf, tsm, rawF],
            [e2t_w[l], e2t_b[l].reshape(1, H),
             tu_w1[l][:4], tu_w1[l][4:], tu_b1[l].reshape(1, H),
             tu_w2[l], tu_b2[l].reshape(1, 4)], 4)

    return _dense_call(
        _dec_body, N, 2000, [nf, uv],
        [dec_w1, dec_b1.reshape(1, H), dec_w2, dec_b2.reshape(1, 2)], 2)


# speedup vs baseline: 3.8183x; 3.6309x over previous
import functools

import jax
import jax.numpy as jnp
from jax import lax
from jax.experimental import pallas as pl
from jax.experimental.pallas import tpu as pltpu
from jax.experimental.pallas import tpu_sc as plsc

N = 50000
E = 400000
T = 200000
H = 32
L = 3


def _dense_call(fn, rows, bn, row_ins, full_ins, out_w):
    grid = rows // bn
    in_specs = (
        [pl.BlockSpec((bn, x.shape[-1]), lambda i: (i, 0)) for x in row_ins]
        + [pl.BlockSpec(x.shape, lambda i, nd=x.ndim: (0,) * nd) for x in full_ins]
    )
    return pl.pallas_call(
        fn,
        grid=(grid,),
        in_specs=in_specs,
        out_specs=pl.BlockSpec((bn, out_w), lambda i: (i, 0)),
        out_shape=jax.ShapeDtypeStruct((rows, out_w), jnp.float32),
    )(*row_ins, *full_ins)


def _enc_body(uv, curv, w1, b1, w2, b2, out):
    h = jnp.tanh(uv[...] @ w1[0:2, :] + curv[...] @ w1[2:4, :] + b1[...])
    out[...] = jnp.tanh(h @ w2[...] + b2[...])


def _node_body(nf, nsm, accD, ndeg4, e2nw, e2nb, w1a, w1b, b1, w2, b2, out):
    cnt = ndeg4[...][:, 0:1]
    nfe = (accD[...] @ e2nw[...] + cnt * e2nb[...]) / jnp.maximum(cnt, 1.0)
    t = jnp.tanh(nf[...] @ w1a[...] + nfe @ w1b[...] + b1[...])
    out[...] = nf[...] + 0.1 * (t @ w2[...] + b2[...]) - 0.05 * nsm[...]


def _edge_body(ef, esm, rawE, accC, tdeg4, n2ew, n2eb, t2ew, t2eb,
               w1a, w1b, b1, w2, b2, out):
    cnt = tdeg4[...][:, 0:1]
    efn = rawE[...] @ n2ew[...] + n2eb[...]
    eft = (accC[...] @ t2ew[...] + cnt * t2eb[...]) / jnp.maximum(cnt, 1.0)
    m = efn + eft
    t = jnp.tanh(ef[...] @ w1a[...] + m @ w1b[...] + b1[...])
    out[...] = ef[...] + 0.1 * (t @ w2[...] + b2[...]) - 0.05 * esm[...]


def _tri_body(tf, tsm, rawF, e2tw, e2tb, w1a, w1b, b1, w2, b2, out):
    tfe = rawF[...] @ e2tw[...] + e2tb[...]
    t = jnp.tanh(tf[...] @ w1a[...] + tfe @ w1b[...] + b1[...])
    out[...] = tf[...] + 0.1 * (t @ w2[...] + b2[...]) - 0.05 * tsm[...]


def _dec_body(nf, uv, w1, b1, w2, b2, out):
    lap = jnp.tanh(nf[...] @ w1[...] + b1[...]) @ w2[...] + b2[...]
    u = uv[...][:, 0:1]
    v = uv[...][:, 1:2]
    ru = (0.1 - u) * (u - 1.0) * u - v
    rv = 0.01 * (0.5 * u - v)
    out[...] = jnp.concatenate(
        [ru + 1e-4 * lap[:, 0:1], rv + 1e-5 * lap[:, 1:2]], axis=-1)


_CH = 128


def _sc_mesh():
    return plsc.VectorSubcoreMesh(core_axis_name="c", subcore_axis_name="s")


_PC = 500


def _acc_zero(acc, tbuf, z_ref, s, npieces):
    pltpu.sync_copy(z_ref, tbuf)

    def piece(j, _):
        g = s + j * 16
        pltpu.sync_copy(tbuf, acc.at[pl.ds(g * _PC, _PC)])
        return _

    lax.fori_loop(0, (npieces + 15 - s) // 16, piece, 0)


def _acc_out(acc, tbuf, out_ref, lo, s, npieces):
    def piece(j, _):
        g = s + j * 16
        pltpu.sync_copy(acc.at[pl.ds(g * _PC, _PC)], tbuf)
        pltpu.sync_copy(tbuf, out_ref.at[pl.ds(lo + g * _PC, _PC)])
        return _

    lax.fori_loop(0, (npieces + 15 - s) // 16, piece, 0)


def _pack_rows(idxv, srcv, dstv, iot, valv=None):
    for g in range(8 * _CH // 16):
        q = g * 16 + iot
        r = q >> 3
        cc = q & 7
        v = plsc.load_gather(idxv, [r])
        d = plsc.load_gather(srcv, [r, cc & 3])
        if valv is not None:
            d = d * plsc.load_gather(valv, [r])
        outv = jnp.where((cc >> 2) == (v & 1), d, 0.0)
        plsc.store_scatter(dstv, [r, cc], outv)


def _remap_packed(idxv, lo, phalf):
    for g in range(_CH // 16):
        v = idxv[pl.ds(g * 16, 16)] >> 1
        ok = (v >= lo) & (v < lo + phalf)
        idxv[pl.ds(g * 16, 16)] = jnp.where(ok, v - lo, phalf)


def _remap(idxv, lo, half):
    for g in range(_CH // 16):
        v = idxv[pl.ds(g * 16, 16)]
        ok = (v >= lo) & (v < lo + half)
        idxv[pl.ds(g * 16, 16)] = jnp.where(ok, v - lo, half)


def _scatter_body(n_idx, K, nout):
    n_chunks = K // _CH
    phalf = nout // 4
    npieces = phalf // _PC

    def body(*refs):
        idx_refs = refs[:n_idx]
        x_ref, z_ref, out_ref = refs[n_idx:n_idx + 3]
        (idxv0, xv0, xv80, idxv1, xv1, xv81,
         tbuf, acc, sem) = refs[n_idx + 3:]
        bufs = ((idxv0, xv0, xv80), (idxv1, xv1, xv81))
        c = lax.axis_index("c")
        s = lax.axis_index("s")
        lo = c * phalf
        iot = lax.iota(jnp.int32, 16)

        _acc_zero(acc, tbuf, z_ref, s, npieces)
        plsc.subcore_barrier()
        n_my = (n_chunks + 15 - s) // 16
        for a in range(n_idx):
            def chunk2(k2, car, a=a):
                for b in range(2):
                    kk = k2 * 2 + b
                    idxv, xv, xv8 = bufs[b]

                    @pl.when(kk < n_my)
                    def _(idxv=idxv, xv=xv, xv8=xv8, kk=kk):
                        off = (s + kk * 16) * _CH
                        pltpu.sync_copy(idx_refs[a].at[pl.ds(off, _CH)], idxv)
                        pltpu.sync_copy(x_ref.at[pl.ds(off, _CH)], xv)
                        _pack_rows(idxv, xv, xv8, iot)
                        _remap_packed(idxv, lo, phalf)
                        pltpu.async_copy(xv8, acc.at[idxv], sem, add=True).wait()
                return car

            lax.fori_loop(0, (n_my + 1) // 2, chunk2, 0)
        plsc.subcore_barrier()
        _acc_out(acc, tbuf, out_ref, lo, s, npieces)

    return body


def _sc_scatter(idxs, x, zeros, nout):
    return pl.kernel(
        _scatter_body(len(idxs), idxs[0].shape[0], nout),
        out_type=jax.ShapeDtypeStruct((nout // 2, 8), jnp.float32),
        mesh=_sc_mesh(),
        compiler_params=pltpu.CompilerParams(use_tc_tiling_on_sc=False, needs_layout_passes=False, internal_scratch_in_bytes=0),
        scratch_types=[
            pltpu.VMEM((_CH,), jnp.int32),
            pltpu.VMEM((_CH, 4), jnp.float32),
            pltpu.VMEM((_CH, 8), jnp.float32),
            pltpu.VMEM((_CH,), jnp.int32),
            pltpu.VMEM((_CH, 4), jnp.float32),
            pltpu.VMEM((_CH, 8), jnp.float32),
            pltpu.VMEM((_PC, 8), jnp.float32),
            pltpu.VMEM_SHARED((nout // 4 + 8, 8), jnp.float32),
            pltpu.SemaphoreType.DMA,
        ],
    )(*idxs, x, zeros)


def _gather_body(n_idx, K, W):
    n_chunks = K // _CH

    def body(*refs):
        idx_refs = refs[:n_idx]
        x_ref = refs[n_idx]
        out_ref = refs[n_idx + 1]
        idxv, gbuf, sem = refs[n_idx + 2:]
        wid = lax.axis_index("s") * 2 + lax.axis_index("c")

        def chunk(k, _):
            off = (wid + k * 32) * _CH
            pltpu.sync_copy(idx_refs[0].at[pl.ds(off, _CH)], idxv)
            pltpu.async_copy(x_ref.at[idxv], gbuf, sem).wait()
            for a in range(1, n_idx):
                pltpu.sync_copy(idx_refs[a].at[pl.ds(off, _CH)], idxv)
                pltpu.async_copy(x_ref.at[idxv], gbuf, sem, add=True).wait()
            pltpu.sync_copy(gbuf, out_ref.at[pl.ds(off, _CH)])
            return _

        lax.fori_loop(0, (n_chunks + 31 - wid) // 32, chunk, 0)

    return body


def _sc_gather_sum(idxs, x):
    K = idxs[0].shape[0]
    W = x.shape[1]
    return pl.kernel(
        _gather_body(len(idxs), K, W),
        out_type=jax.ShapeDtypeStruct((K, W), jnp.float32),
        mesh=_sc_mesh(),
        compiler_params=pltpu.CompilerParams(use_tc_tiling_on_sc=False, needs_layout_passes=False, internal_scratch_in_bytes=0),
        scratch_types=[
            pltpu.VMEM((_CH,), jnp.int32),
            pltpu.VMEM((_CH, W), jnp.float32),
            pltpu.SemaphoreType.DMA,
        ],
    )(*idxs, x)


def _scatter32_body(K, nout):
    n_chunks = K // _CH
    half = nout // 2
    npieces = half // _PC

    def body(rows_ref, x_ref, z_ref, out_ref, rowv0, xv0, rowv1, xv1,
             tbuf, acc, sem):
        bufs = ((rowv0, xv0), (rowv1, xv1))
        c = lax.axis_index("c")
        s = lax.axis_index("s")
        lo = c * half

        _acc_zero(acc, tbuf, z_ref, s, npieces)
        plsc.subcore_barrier()
        n_my = (n_chunks + 15 - s) // 16

        def chunk2(k2, car):
            for b in range(2):
                kk = k2 * 2 + b
                rowv, xv = bufs[b]

                @pl.when(kk < n_my)
                def _(rowv=rowv, xv=xv, kk=kk):
                    off = (s + kk * 16) * _CH
                    pltpu.sync_copy(rows_ref.at[pl.ds(off, _CH)], rowv)
                    pltpu.sync_copy(x_ref.at[pl.ds(off, _CH)], xv)
                    _remap(rowv, lo, half)
                    pltpu.async_copy(xv, acc.at[rowv], sem, add=True).wait()
            return car

        lax.fori_loop(0, (n_my + 1) // 2, chunk2, 0)
        plsc.subcore_barrier()
        _acc_out(acc, tbuf, out_ref, lo, s, npieces)

    return body


def _sc_scatter32(rows, x, zeros, nout):
    return pl.kernel(
        _scatter32_body(rows.shape[0], nout),
        out_type=jax.ShapeDtypeStruct((nout, 32), jnp.float32),
        mesh=_sc_mesh(),
        compiler_params=pltpu.CompilerParams(use_tc_tiling_on_sc=False, needs_layout_passes=False, internal_scratch_in_bytes=0),
        scratch_types=[
            pltpu.VMEM((_CH,), jnp.int32),
            pltpu.VMEM((_CH, 32), jnp.float32),
            pltpu.VMEM((_CH,), jnp.int32),
            pltpu.VMEM((_CH, 32), jnp.float32),
            pltpu.VMEM((_PC, 32), jnp.float32),
            pltpu.VMEM_SHARED((nout // 2 + 8, 32), jnp.float32),
            pltpu.SemaphoreType.DMA,
        ],
    )(rows, x, zeros)


def _scale_body(g, v, out):
    out[...] = g[...] * v[...]


def _spmm4_body(K, nout):
    n_chunks = K // _CH
    phalf = nout // 4
    npieces = phalf // _PC

    def body(rows_ref, cols_ref, vals_ref, x_ref, z_ref, out_ref,
             rowv0, colv0, valv0, gbuf0, sbuf0,
             rowv1, colv1, valv1, gbuf1, sbuf1,
             tbuf, acc, semg, sems):
        bufs = ((rowv0, colv0, valv0, gbuf0, sbuf0),
                (rowv1, colv1, valv1, gbuf1, sbuf1))
        c = lax.axis_index("c")
        s = lax.axis_index("s")
        lo = c * phalf
        iot = lax.iota(jnp.int32, 16)

        _acc_zero(acc, tbuf, z_ref, s, npieces)
        plsc.subcore_barrier()
        n_my = (n_chunks + 15 - s) // 16

        def chunk2(k2, car):
            for b in range(2):
                kk = k2 * 2 + b
                rowv, colv, valv, gbuf, sbuf = bufs[b]

                @pl.when(kk < n_my)
                def _(rowv=rowv, colv=colv, valv=valv, gbuf=gbuf,
                      sbuf=sbuf, kk=kk):
                    off = (s + kk * 16) * _CH
                    pltpu.sync_copy(cols_ref.at[pl.ds(off, _CH)], colv)
                    pltpu.async_copy(x_ref.at[colv], gbuf, semg).wait()
                    pltpu.sync_copy(vals_ref.at[pl.ds(off, _CH)], valv)
                    pltpu.sync_copy(rows_ref.at[pl.ds(off, _CH)], rowv)
                    _pack_rows(rowv, gbuf, sbuf, iot, valv=valv)
                    _remap_packed(rowv, lo, phalf)
                    pltpu.async_copy(sbuf, acc.at[rowv], sems, add=True).wait()
            return car

        lax.fori_loop(0, (n_my + 1) // 2, chunk2, 0)
        plsc.subcore_barrier()
        _acc_out(acc, tbuf, out_ref, lo, s, npieces)

    return body


def _sc_spmm32(rows, cols, vals, x, zeros, nout):
    return pl.kernel(
        _spmm32_body(rows.shape[0], nout),
        out_type=jax.ShapeDtypeStruct((nout, 32), jnp.float32),
        mesh=_sc_mesh(),
        compiler_params=pltpu.CompilerParams(use_tc_tiling_on_sc=False, needs_layout_passes=False, internal_scratch_in_bytes=0),
        scratch_types=[
            pltpu.VMEM((_CH,), jnp.int32),
            pltpu.VMEM((_CH,), jnp.int32),
            pltpu.VMEM((_CH,), jnp.float32),
            pltpu.VMEM((_CH, 32), jnp.float32),
            pltpu.VMEM((_CH, 32), jnp.float32),
            pltpu.VMEM((_CH,), jnp.int32),
            pltpu.VMEM((_CH,), jnp.int32),
            pltpu.VMEM((_CH,), jnp.float32),
            pltpu.VMEM((_CH, 32), jnp.float32),
            pltpu.VMEM((_CH, 32), jnp.float32),
            pltpu.VMEM((_PC, 32), jnp.float32),
            pltpu.VMEM_SHARED((nout // 2 + 8, 32), jnp.float32),
            pltpu.SemaphoreType.DMA,
            pltpu.SemaphoreType.DMA,
        ],
    )(rows, cols, vals, x, zeros)


def _sc_spmm4(rows, cols, vals, x16, zeros, nout):
    return pl.kernel(
        _spmm4_body(rows.shape[0], nout),
        out_type=jax.ShapeDtypeStruct((nout // 2, 8), jnp.float32),
        mesh=_sc_mesh(),
        compiler_params=pltpu.CompilerParams(use_tc_tiling_on_sc=False, needs_layout_passes=False, internal_scratch_in_bytes=0),
        scratch_types=[
            pltpu.VMEM((_CH,), jnp.int32),
            pltpu.VMEM((_CH,), jnp.int32),
            pltpu.VMEM((_CH,), jnp.float32),
            pltpu.VMEM((_CH, 16), jnp.float32),
            pltpu.VMEM((_CH, 8), jnp.float32),
            pltpu.VMEM((_CH,), jnp.int32),
            pltpu.VMEM((_CH,), jnp.int32),
            pltpu.VMEM((_CH,), jnp.float32),
            pltpu.VMEM((_CH, 16), jnp.float32),
            pltpu.VMEM((_CH, 8), jnp.float32),
            pltpu.VMEM((_PC, 8), jnp.float32),
            pltpu.VMEM_SHARED((nout // 4 + 8, 8), jnp.float32),
            pltpu.SemaphoreType.DMA,
            pltpu.SemaphoreType.DMA,
        ],
    )(rows, cols, vals, x16, zeros)


def _pad_to(a, n, v):
    pad_n = n - a.shape[0]
    return jnp.concatenate(
        [a, jnp.full((pad_n,) + a.shape[1:], v, a.dtype)], axis=0)


def kernel(uv, curvatures, edge_feat, tri_feat, edge_index, tri_edges,
           rows0, cols0, vals0, rows1, cols1, vals1, rows2, cols2, vals2,
           enc_w1, enc_b1, enc_w2, enc_b2, nu_w1, nu_b1, nu_w2, nu_b2,
           eu_w1, eu_b1, eu_w2, eu_b2, tu_w1, tu_b1, tu_w2, tu_b2,
           e2n_w, e2n_b, n2e_w, n2e_b, t2e_w, t2e_b, e2t_w, e2t_b,
           dec_w1, dec_b1, dec_w2, dec_b2):
    src = edge_index[0]
    dst = edge_index[1]
    NNZ0 = rows0.shape[0]
    NNZ0p = ((NNZ0 + _CH - 1) // _CH) * _CH
    T_ = tri_edges.shape[0]
    Tp = ((T_ + _CH - 1) // _CH) * _CH

    te0 = _pad_to(tri_edges[:, 0], Tp, 0)
    te1 = _pad_to(tri_edges[:, 1], Tp, 0)
    te2 = _pad_to(tri_edges[:, 2], Tp, 0)
    rows0p = _pad_to(rows0, NNZ0p, 0)
    cols0p = _pad_to(cols0, NNZ0p, 0)
    vals0p = _pad_to(vals0, NNZ0p, 0.0)

    z32 = jnp.zeros((_PC, H), jnp.float32)
    z8 = jnp.zeros((_PC, 8), jnp.float32)
    onesE4 = jnp.ones((E, 4), jnp.float32)
    onesT4 = _pad_to(jnp.ones((T, 4), jnp.float32), Tp, 0.0)

    ndeg4 = _sc_scatter([src, dst], onesE4, z8, N).reshape(N, 4)
    tdeg4 = _sc_scatter([te0, te1, te2], onesT4, z8, E).reshape(E, 4)

    nf = _dense_call(_enc_body, N, 2000, [uv, curvatures],
                     [enc_w1, enc_b1.reshape(1, H), enc_w2, enc_b2.reshape(1, H)], H)
    ef = edge_feat
    tf = tri_feat

    for l in range(L):
        tfp = _pad_to(tf, Tp, 0.0)
        ef16 = jnp.pad(ef, ((0, 0), (0, 12)))
        tf16 = jnp.pad(tf, ((0, 0), (0, 12)))
        nfg = _sc_gather_sum([cols0p], nf)
        nfs = _dense_call(_scale_body, NNZ0p, 3712,
                          [nfg, vals0p.reshape(NNZ0p, 1)], [], H)
        nsm = _sc_scatter32(rows0p, nfs, z32, N)
        esm = _sc_spmm4(rows1, cols1, vals1, ef16, z8, E).reshape(E, 4)
        tsm = _sc_spmm4(rows2, cols2, vals2, tf16, z8, T).reshape(T, 4)
        accD = _sc_scatter([src, dst], ef, z8, N).reshape(N, 4)
        accC = _sc_scatter([te0, te1, te2], tfp, z8, E).reshape(E, 4)
        rawE = _sc_gather_sum([src, dst], nf)
        rawF = _sc_gather_sum([te0, te1, te2], ef16)[:T]

        nf = _dense_call(
            _node_body, N, 2000, [nf, nsm, accD, ndeg4],
            [e2n_w[l], e2n_b[l].reshape(1, H),
             nu_w1[l][:H], nu_w1[l][H:], nu_b1[l].reshape(1, H),
             nu_w2[l], nu_b2[l].reshape(1, H)], H)
        ef = _dense_call(
            _edge_body, E, 2000, [ef, esm, rawE, accC, tdeg4],
            [n2e_w[l] * 0.5, n2e_b[l].reshape(1, H),
             t2e_w[l], t2e_b[l].reshape(1, H),
             eu_w1[l][:4], eu_w1[l][4:], eu_b1[l].reshape(1, H),
             eu_w2[l], eu_b2[l].reshape(1, 4)], 4)
        tf = _dense_call(
            _tri_body, T, 2000, [tf, tsm, rawF],
            [jnp.concatenate([e2t_w[l] * (1.0 / 3.0),
                              jnp.zeros((12, H), jnp.float32)]),
             e2t_b[l].reshape(1, H),
             tu_w1[l][:4], tu_w1[l][4:], tu_b1[l].reshape(1, H),
             tu_w2[l], tu_b2[l].reshape(1, 4)], 4)

    return _dense_call(
        _dec_body, N, 2000, [nf, uv],
        [dec_w1, dec_b1.reshape(1, H), dec_w2, dec_b2.reshape(1, 2)], 2)


# speedup vs baseline: 4.4236x; 1.1585x over previous
import functools

import jax
import jax.numpy as jnp
from jax import lax
from jax.experimental import pallas as pl
from jax.experimental.pallas import tpu as pltpu
from jax.experimental.pallas import tpu_sc as plsc

N = 50000
E = 400000
T = 200000
H = 32
L = 3


def _dense_call(fn, rows, bn, row_ins, full_ins, out_w):
    grid = rows // bn
    in_specs = (
        [pl.BlockSpec((bn, x.shape[-1]), lambda i: (i, 0)) for x in row_ins]
        + [pl.BlockSpec(x.shape, lambda i, nd=x.ndim: (0,) * nd) for x in full_ins]
    )
    return pl.pallas_call(
        fn,
        grid=(grid,),
        in_specs=in_specs,
        out_specs=pl.BlockSpec((bn, out_w), lambda i: (i, 0)),
        out_shape=jax.ShapeDtypeStruct((rows, out_w), jnp.float32),
    )(*row_ins, *full_ins)


def _enc_body(uv, curv, w1, b1, w2, b2, out):
    h = jnp.tanh(uv[...] @ w1[0:2, :] + curv[...] @ w1[2:4, :] + b1[...])
    out[...] = jnp.tanh(h @ w2[...] + b2[...])


def _node_body(nf, nsm, accD, ndeg4, e2nw, e2nb, w1a, w1b, b1, w2, b2, out):
    cnt = ndeg4[...][:, 0:1]
    nfe = (accD[...] @ e2nw[...] + cnt * e2nb[...]) / jnp.maximum(cnt, 1.0)
    t = jnp.tanh(nf[...] @ w1a[...] + nfe @ w1b[...] + b1[...])
    out[...] = nf[...] + 0.1 * (t @ w2[...] + b2[...]) - 0.05 * nsm[...]


def _edge_body(ef, esm, rawE, accC, tdeg4, n2ew, n2eb, t2ew, t2eb,
               w1a, w1b, b1, w2, b2, out):
    cnt = tdeg4[...][:, 0:1]
    efn = rawE[...] @ n2ew[...] + n2eb[...]
    eft = (accC[...] @ t2ew[...] + cnt * t2eb[...]) / jnp.maximum(cnt, 1.0)
    m = efn + eft
    t = jnp.tanh(ef[...] @ w1a[...] + m @ w1b[...] + b1[...])
    out[...] = ef[...] + 0.1 * (t @ w2[...] + b2[...]) - 0.05 * esm[...]


def _tri_body(tf, tsm, rawF, e2tw, e2tb, w1a, w1b, b1, w2, b2, out):
    tfe = rawF[...] @ e2tw[...] + e2tb[...]
    t = jnp.tanh(tf[...] @ w1a[...] + tfe @ w1b[...] + b1[...])
    out[...] = tf[...] + 0.1 * (t @ w2[...] + b2[...]) - 0.05 * tsm[...]


def _dec_body(nf, uv, w1, b1, w2, b2, out):
    lap = jnp.tanh(nf[...] @ w1[...] + b1[...]) @ w2[...] + b2[...]
    u = uv[...][:, 0:1]
    v = uv[...][:, 1:2]
    ru = (0.1 - u) * (u - 1.0) * u - v
    rv = 0.01 * (0.5 * u - v)
    out[...] = jnp.concatenate(
        [ru + 1e-4 * lap[:, 0:1], rv + 1e-5 * lap[:, 1:2]], axis=-1)


_CH = 128


def _sc_mesh():
    return plsc.VectorSubcoreMesh(core_axis_name="c", subcore_axis_name="s")


_PC = 500


def _acc_zero(acc, tbuf, z_ref, s, npieces):
    pltpu.sync_copy(z_ref, tbuf)

    def piece(j, _):
        g = s + j * 16
        pltpu.sync_copy(tbuf, acc.at[pl.ds(g * _PC, _PC)])
        return _

    lax.fori_loop(0, (npieces + 15 - s) // 16, piece, 0)


def _acc_out(acc, tbuf, out_ref, lo, s, npieces):
    def piece(j, _):
        g = s + j * 16
        pltpu.sync_copy(acc.at[pl.ds(g * _PC, _PC)], tbuf)
        pltpu.sync_copy(tbuf, out_ref.at[pl.ds(lo + g * _PC, _PC)])
        return _

    lax.fori_loop(0, (npieces + 15 - s) // 16, piece, 0)


def _pack_rows(idxv, srcv, dstv, iot, valv=None):
    for g in range(8 * _CH // 16):
        q = g * 16 + iot
        r = q >> 3
        cc = q & 7
        v = plsc.load_gather(idxv, [r])
        d = plsc.load_gather(srcv, [r, cc & 3])
        if valv is not None:
            d = d * plsc.load_gather(valv, [r])
        outv = jnp.where((cc >> 2) == (v & 1), d, 0.0)
        plsc.store_scatter(dstv, [r, cc], outv)


def _remap_packed(idxv, lo, phalf):
    for g in range(_CH // 16):
        v = idxv[pl.ds(g * 16, 16)] >> 1
        ok = (v >= lo) & (v < lo + phalf)
        idxv[pl.ds(g * 16, 16)] = jnp.where(ok, v - lo, phalf)


def _remap(idxv, lo, half):
    for g in range(_CH // 16):
        v = idxv[pl.ds(g * 16, 16)]
        ok = (v >= lo) & (v < lo + half)
        idxv[pl.ds(g * 16, 16)] = jnp.where(ok, v - lo, half)


def _scatter_body(n_idx, K, nout):
    n_chunks = K // _CH
    phalf = nout // 4
    npieces = phalf // _PC

    def body(*refs):
        idx_refs = refs[:n_idx]
        x_ref, z_ref, out_ref = refs[n_idx:n_idx + 3]
        (idxv0, xv0, xv80, idxv1, xv1, xv81,
         tbuf, acc, sem) = refs[n_idx + 3:]
        bufs = ((idxv0, xv0, xv80), (idxv1, xv1, xv81))
        c = lax.axis_index("c")
        s = lax.axis_index("s")
        lo = c * phalf
        iot = lax.iota(jnp.int32, 16)

        _acc_zero(acc, tbuf, z_ref, s, npieces)
        plsc.subcore_barrier()
        n_my = (n_chunks + 15 - s) // 16
        for a in range(n_idx):
            def chunk2(k2, car, a=a):
                for b in range(2):
                    kk = k2 * 2 + b
                    idxv, xv, xv8 = bufs[b]

                    @pl.when(kk < n_my)
                    def _(idxv=idxv, xv=xv, xv8=xv8, kk=kk):
                        off = (s + kk * 16) * _CH
                        pltpu.sync_copy(idx_refs[a].at[pl.ds(off, _CH)], idxv)
                        pltpu.sync_copy(x_ref.at[pl.ds(off, _CH)], xv)
                        _pack_rows(idxv, xv, xv8, iot)
                        _remap_packed(idxv, lo, phalf)
                        pltpu.async_copy(xv8, acc.at[idxv], sem, add=True).wait()
                return car

            lax.fori_loop(0, (n_my + 1) // 2, chunk2, 0)
        plsc.subcore_barrier()
        _acc_out(acc, tbuf, out_ref, lo, s, npieces)

    return body


def _sc_scatter(idxs, x, zeros, nout):
    return pl.kernel(
        _scatter_body(len(idxs), idxs[0].shape[0], nout),
        out_type=jax.ShapeDtypeStruct((nout // 2, 8), jnp.float32),
        mesh=_sc_mesh(),
        compiler_params=pltpu.CompilerParams(use_tc_tiling_on_sc=False, needs_layout_passes=False, internal_scratch_in_bytes=0),
        scratch_types=[
            pltpu.VMEM((_CH,), jnp.int32),
            pltpu.VMEM((_CH, 4), jnp.float32),
            pltpu.VMEM((_CH, 8), jnp.float32),
            pltpu.VMEM((_CH,), jnp.int32),
            pltpu.VMEM((_CH, 4), jnp.float32),
            pltpu.VMEM((_CH, 8), jnp.float32),
            pltpu.VMEM((_PC, 8), jnp.float32),
            pltpu.VMEM_SHARED((nout // 4 + 8, 8), jnp.float32),
            pltpu.SemaphoreType.DMA,
        ],
    )(*idxs, x, zeros)


def _gather_body(n_idx, K, W):
    n_chunks = K // _CH

    def body(*refs):
        idx_refs = refs[:n_idx]
        x_ref = refs[n_idx]
        out_ref = refs[n_idx + 1]
        idxv, gbuf, sem = refs[n_idx + 2:]
        wid = lax.axis_index("s") * 2 + lax.axis_index("c")

        def chunk(k, _):
            off = (wid + k * 32) * _CH
            pltpu.sync_copy(idx_refs[0].at[pl.ds(off, _CH)], idxv)
            pltpu.async_copy(x_ref.at[idxv], gbuf, sem).wait()
            for a in range(1, n_idx):
                pltpu.sync_copy(idx_refs[a].at[pl.ds(off, _CH)], idxv)
                pltpu.async_copy(x_ref.at[idxv], gbuf, sem, add=True).wait()
            pltpu.sync_copy(gbuf, out_ref.at[pl.ds(off, _CH)])
            return _

        lax.fori_loop(0, (n_chunks + 31 - wid) // 32, chunk, 0)

    return body


def _sc_gather_sum(idxs, x):
    K = idxs[0].shape[0]
    W = x.shape[1]
    return pl.kernel(
        _gather_body(len(idxs), K, W),
        out_type=jax.ShapeDtypeStruct((K, W), jnp.float32),
        mesh=_sc_mesh(),
        compiler_params=pltpu.CompilerParams(use_tc_tiling_on_sc=False, needs_layout_passes=False, internal_scratch_in_bytes=0),
        scratch_types=[
            pltpu.VMEM((_CH,), jnp.int32),
            pltpu.VMEM((_CH, W), jnp.float32),
            pltpu.SemaphoreType.DMA,
        ],
    )(*idxs, x)


def _scatter32_body(K, nout):
    n_chunks = K // _CH
    half = nout // 2
    npieces = half // _PC

    def body(rows_ref, x_ref, z_ref, out_ref, rowv0, xv0, rowv1, xv1,
             tbuf, acc, sem):
        bufs = ((rowv0, xv0), (rowv1, xv1))
        c = lax.axis_index("c")
        s = lax.axis_index("s")
        lo = c * half

        _acc_zero(acc, tbuf, z_ref, s, npieces)
        plsc.subcore_barrier()
        n_my = (n_chunks + 15 - s) // 16

        def chunk2(k2, car):
            for b in range(2):
                kk = k2 * 2 + b
                rowv, xv = bufs[b]

                @pl.when(kk < n_my)
                def _(rowv=rowv, xv=xv, kk=kk):
                    off = (s + kk * 16) * _CH
                    pltpu.sync_copy(rows_ref.at[pl.ds(off, _CH)], rowv)
                    pltpu.sync_copy(x_ref.at[pl.ds(off, _CH)], xv)
                    _remap(rowv, lo, half)
                    pltpu.async_copy(xv, acc.at[rowv], sem, add=True).wait()
            return car

        lax.fori_loop(0, (n_my + 1) // 2, chunk2, 0)
        plsc.subcore_barrier()
        _acc_out(acc, tbuf, out_ref, lo, s, npieces)

    return body


def _sc_scatter32(rows, x, zeros, nout):
    return pl.kernel(
        _scatter32_body(rows.shape[0], nout),
        out_type=jax.ShapeDtypeStruct((nout, 32), jnp.float32),
        mesh=_sc_mesh(),
        compiler_params=pltpu.CompilerParams(use_tc_tiling_on_sc=False, needs_layout_passes=False, internal_scratch_in_bytes=0),
        scratch_types=[
            pltpu.VMEM((_CH,), jnp.int32),
            pltpu.VMEM((_CH, 32), jnp.float32),
            pltpu.VMEM((_CH,), jnp.int32),
            pltpu.VMEM((_CH, 32), jnp.float32),
            pltpu.VMEM((_PC, 32), jnp.float32),
            pltpu.VMEM_SHARED((nout // 2 + 8, 32), jnp.float32),
            pltpu.SemaphoreType.DMA,
        ],
    )(rows, x, zeros)


def _scale_body(g, v, out):
    out[...] = g[...] * v[...]


def _spmm4_body(K, nout):
    n_chunks = K // _CH
    phalf = nout // 4
    npieces = phalf // _PC

    def body(rows_ref, cols_ref, vals_ref, x_ref, z_ref, out_ref,
             rowv0, colv0, valv0, gbuf0, sbuf0,
             rowv1, colv1, valv1, gbuf1, sbuf1,
             tbuf, acc, semg, sems, semi):
        bufs = ((rowv0, colv0, valv0, gbuf0, sbuf0),
                (rowv1, colv1, valv1, gbuf1, sbuf1))
        c = lax.axis_index("c")
        s = lax.axis_index("s")
        lo = c * phalf
        iot = lax.iota(jnp.int32, 16)

        _acc_zero(acc, tbuf, z_ref, s, npieces)
        plsc.subcore_barrier()
        n_my = (n_chunks + 15 - s) // 16

        def chunk2(k2, car):
            for b in range(2):
                kk = k2 * 2 + b
                rowv, colv, valv, gbuf, sbuf = bufs[b]

                @pl.when(kk < n_my)
                def _(rowv=rowv, colv=colv, valv=valv, gbuf=gbuf,
                      sbuf=sbuf, kk=kk):
                    off = (s + kk * 16) * _CH
                    dc = pltpu.async_copy(cols_ref.at[pl.ds(off, _CH)], colv, semi)
                    dv = pltpu.async_copy(vals_ref.at[pl.ds(off, _CH)], valv, semg)
                    dr = pltpu.async_copy(rows_ref.at[pl.ds(off, _CH)], rowv, semg)
                    dc.wait()
                    dg = pltpu.async_copy(x_ref.at[colv], gbuf, semg)
                    dv.wait()
                    dr.wait()
                    dg.wait()
                    _pack_rows(rowv, gbuf, sbuf, iot, valv=valv)
                    _remap_packed(rowv, lo, phalf)
                    pltpu.async_copy(sbuf, acc.at[rowv], sems, add=True).wait()
            return car

        lax.fori_loop(0, (n_my + 1) // 2, chunk2, 0)
        plsc.subcore_barrier()
        _acc_out(acc, tbuf, out_ref, lo, s, npieces)

    return body


def _sc_spmm32(rows, cols, vals, x, zeros, nout):
    return pl.kernel(
        _spmm32_body(rows.shape[0], nout),
        out_type=jax.ShapeDtypeStruct((nout, 32), jnp.float32),
        mesh=_sc_mesh(),
        compiler_params=pltpu.CompilerParams(use_tc_tiling_on_sc=False, needs_layout_passes=False, internal_scratch_in_bytes=0),
        scratch_types=[
            pltpu.VMEM((_CH,), jnp.int32),
            pltpu.VMEM((_CH,), jnp.int32),
            pltpu.VMEM((_CH,), jnp.float32),
            pltpu.VMEM((_CH, 32), jnp.float32),
            pltpu.VMEM((_CH, 32), jnp.float32),
            pltpu.VMEM((_CH,), jnp.int32),
            pltpu.VMEM((_CH,), jnp.int32),
            pltpu.VMEM((_CH,), jnp.float32),
            pltpu.VMEM((_CH, 32), jnp.float32),
            pltpu.VMEM((_CH, 32), jnp.float32),
            pltpu.VMEM((_PC, 32), jnp.float32),
            pltpu.VMEM_SHARED((nout // 2 + 8, 32), jnp.float32),
            pltpu.SemaphoreType.DMA,
            pltpu.SemaphoreType.DMA,
        ],
    )(rows, cols, vals, x, zeros)


def _sc_spmm4(rows, cols, vals, x16, zeros, nout):
    return pl.kernel(
        _spmm4_body(rows.shape[0], nout),
        out_type=jax.ShapeDtypeStruct((nout // 2, 8), jnp.float32),
        mesh=_sc_mesh(),
        compiler_params=pltpu.CompilerParams(use_tc_tiling_on_sc=False, needs_layout_passes=False, internal_scratch_in_bytes=0),
        scratch_types=[
            pltpu.VMEM((_CH,), jnp.int32),
            pltpu.VMEM((_CH,), jnp.int32),
            pltpu.VMEM((_CH,), jnp.float32),
            pltpu.VMEM((_CH, 16), jnp.float32),
            pltpu.VMEM((_CH, 8), jnp.float32),
            pltpu.VMEM((_CH,), jnp.int32),
            pltpu.VMEM((_CH,), jnp.int32),
            pltpu.VMEM((_CH,), jnp.float32),
            pltpu.VMEM((_CH, 16), jnp.float32),
            pltpu.VMEM((_CH, 8), jnp.float32),
            pltpu.VMEM((_PC, 8), jnp.float32),
            pltpu.VMEM_SHARED((nout // 4 + 8, 8), jnp.float32),
            pltpu.SemaphoreType.DMA,
            pltpu.SemaphoreType.DMA,
            pltpu.SemaphoreType.DMA,
        ],
    )(rows, cols, vals, x16, zeros)


def _pad_to(a, n, v):
    pad_n = n - a.shape[0]
    return jnp.concatenate(
        [a, jnp.full((pad_n,) + a.shape[1:], v, a.dtype)], axis=0)


def kernel(uv, curvatures, edge_feat, tri_feat, edge_index, tri_edges,
           rows0, cols0, vals0, rows1, cols1, vals1, rows2, cols2, vals2,
           enc_w1, enc_b1, enc_w2, enc_b2, nu_w1, nu_b1, nu_w2, nu_b2,
           eu_w1, eu_b1, eu_w2, eu_b2, tu_w1, tu_b1, tu_w2, tu_b2,
           e2n_w, e2n_b, n2e_w, n2e_b, t2e_w, t2e_b, e2t_w, e2t_b,
           dec_w1, dec_b1, dec_w2, dec_b2):
    src = edge_index[0]
    dst = edge_index[1]
    NNZ0 = rows0.shape[0]
    NNZ0p = ((NNZ0 + _CH - 1) // _CH) * _CH
    T_ = tri_edges.shape[0]
    Tp = ((T_ + _CH - 1) // _CH) * _CH

    te0 = _pad_to(tri_edges[:, 0], Tp, 0)
    te1 = _pad_to(tri_edges[:, 1], Tp, 0)
    te2 = _pad_to(tri_edges[:, 2], Tp, 0)
    rows0p = _pad_to(rows0, NNZ0p, 0)
    cols0p = _pad_to(cols0, NNZ0p, 0)
    vals0p = _pad_to(vals0, NNZ0p, 0.0)

    z32 = jnp.zeros((_PC, H), jnp.float32)
    z8 = jnp.zeros((_PC, 8), jnp.float32)
    onesE4 = jnp.ones((E, 4), jnp.float32)
    onesT4 = _pad_to(jnp.ones((T, 4), jnp.float32), Tp, 0.0)

    ndeg4 = _sc_scatter([src, dst], onesE4, z8, N).reshape(N, 4)
    tdeg4 = _sc_scatter([te0, te1, te2], onesT4, z8, E).reshape(E, 4)

    nf = _dense_call(_enc_body, N, 2000, [uv, curvatures],
                     [enc_w1, enc_b1.reshape(1, H), enc_w2, enc_b2.reshape(1, H)], H)
    ef = edge_feat
    tf = tri_feat

    for l in range(L):
        tfp = _pad_to(tf, Tp, 0.0)
        ef16 = jnp.pad(ef, ((0, 0), (0, 12)))
        tf16 = jnp.pad(tf, ((0, 0), (0, 12)))
        nfg = _sc_gather_sum([cols0p], nf)
        nfs = _dense_call(_scale_body, NNZ0p, 3712,
                          [nfg, vals0p.reshape(NNZ0p, 1)], [], H)
        nsm = _sc_scatter32(rows0p, nfs, z32, N)
        esm = _sc_spmm4(rows1, cols1, vals1, ef16, z8, E).reshape(E, 4)
        tsm = _sc_spmm4(rows2, cols2, vals2, tf16, z8, T).reshape(T, 4)
        accD = _sc_scatter([src, dst], ef, z8, N).reshape(N, 4)
        accC = _sc_scatter([te0, te1, te2], tfp, z8, E).reshape(E, 4)
        rawE = _sc_gather_sum([src, dst], nf)
        rawF = _sc_gather_sum([te0, te1, te2], ef16)[:T]

        nf = _dense_call(
            _node_body, N, 2000, [nf, nsm, accD, ndeg4],
            [e2n_w[l], e2n_b[l].reshape(1, H),
             nu_w1[l][:H], nu_w1[l][H:], nu_b1[l].reshape(1, H),
             nu_w2[l], nu_b2[l].reshape(1, H)], H)
        ef = _dense_call(
            _edge_body, E, 2000, [ef, esm, rawE, accC, tdeg4],
            [n2e_w[l] * 0.5, n2e_b[l].reshape(1, H),
             t2e_w[l], t2e_b[l].reshape(1, H),
             eu_w1[l][:4], eu_w1[l][4:], eu_b1[l].reshape(1, H),
             eu_w2[l], eu_b2[l].reshape(1, 4)], 4)
        tf = _dense_call(
            _tri_body, T, 2000, [tf, tsm, rawF],
            [jnp.concatenate([e2t_w[l] * (1.0 / 3.0),
                              jnp.zeros((12, H), jnp.float32)]),
             e2t_b[l].reshape(1, H),
             tu_w1[l][:4], tu_w1[l][4:], tu_b1[l].reshape(1, H),
             tu_w2[l], tu_b2[l].reshape(1, 4)], 4)

    return _dense_call(
        _dec_body, N, 2000, [nf, uv],
        [dec_w1, dec_b1.reshape(1, H), dec_w2, dec_b2.reshape(1, 2)], 2)
